# Initial kernel scaffold; baseline (speedup 1.0000x reference)
import jax
import jax.numpy as jnp
from jax.experimental import pallas as pl


def kernel(x, emb, W, b):
    raise NotImplementedError("write your pallas kernel here")


# speedup vs baseline: 113.4585x; 113.4585x over previous
import functools

import jax
import jax.numpy as jnp
from jax import lax
from jax.experimental import pallas as pl
from jax.experimental.pallas import tpu as pltpu
from jax.experimental.pallas import tpu_sc as plsc


def _table_body(s, emb_ref, w_ref, b_ref, t_ref):
    t_ref[...] = (
        jnp.sum(emb_ref[...] * w_ref[...], axis=1, keepdims=True) + b_ref[0]
    ) * (1.0 / s)


@functools.lru_cache(maxsize=None)
def _make_table_kernel(v_dim, d_dim, s_len):
    return pl.pallas_call(
        functools.partial(_table_body, s_len),
        out_shape=jax.ShapeDtypeStruct((v_dim, 1), jnp.float32),
        in_specs=[
            pl.BlockSpec(memory_space=pltpu.VMEM),
            pl.BlockSpec(memory_space=pltpu.VMEM),
            pl.BlockSpec(memory_space=pltpu.SMEM),
        ],
        out_specs=pl.BlockSpec(memory_space=pltpu.VMEM),
    )


@functools.lru_cache(maxsize=None)
def _make_sc_kernel(b_rows, s_len, v_dim):
    info = plsc.get_sparse_core_info()
    nc, ns, lanes = info.num_cores, info.num_subcores, info.num_lanes
    nw = nc * ns
    rows_per_w = b_rows // nw
    group = lanes
    n_groups = rows_per_w // group

    mesh = plsc.VectorSubcoreMesh(core_axis_name="c", subcore_axis_name="s")

    @functools.partial(
        pl.kernel,
        mesh=mesh,
        out_type=jax.ShapeDtypeStruct((b_rows,), jnp.float32),
        scratch_types=[
            pltpu.VMEM((group * s_len,), jnp.int32),
            pltpu.VMEM((v_dim,), jnp.float32),
            pltpu.VMEM((rows_per_w,), jnp.float32),
        ],
        compiler_params=pltpu.CompilerParams(needs_layout_passes=False),
    )
    def sc_kernel(x_hbm, t_hbm, out_hbm, x_v, t_v, o_v):
        wid = lax.axis_index("s") * nc + lax.axis_index("c")
        pltpu.sync_copy(t_hbm, t_v)
        lane_base = lax.iota(jnp.int32, lanes) * s_len
        out_idx = lax.iota(jnp.int32, lanes)

        def do_group(g, carry):
            off = wid * (rows_per_w * s_len) + g * (group * s_len)
            pltpu.sync_copy(x_hbm.at[pl.ds(off, group * s_len)], x_v)

            def step(s, acc):
                xi = plsc.load_gather(x_v, [lane_base + s])
                return acc + plsc.load_gather(t_v, [xi])

            acc = lax.fori_loop(0, s_len, step, jnp.zeros((lanes,), jnp.float32))
            res = 1.0 / (1.0 + jnp.exp(-acc))
            plsc.store_scatter(o_v, [g * group + out_idx], res)
            return carry

        lax.fori_loop(0, n_groups, do_group, 0)
        pltpu.sync_copy(o_v, out_hbm.at[pl.ds(wid * rows_per_w, rows_per_w)])

    return sc_kernel


def kernel(x, emb, W, b):
    b_rows, s_len = x.shape
    v_dim, d_dim = emb.shape
    t = _make_table_kernel(v_dim, d_dim, s_len)(emb, W, b)
    out = _make_sc_kernel(b_rows, s_len, v_dim)(x.reshape(-1), t.reshape(-1))
    return out.reshape(b_rows, 1)


# speedup vs baseline: 193.2245x; 1.7030x over previous
import functools

import jax
import jax.numpy as jnp
from jax import lax
from jax.experimental import pallas as pl
from jax.experimental.pallas import tpu as pltpu
from jax.experimental.pallas import tpu_sc as plsc


def _table_body(s, emb_ref, w_ref, b_ref, t_ref):
    t_ref[...] = (
        jnp.sum(emb_ref[...] * w_ref[...], axis=1, keepdims=True) + b_ref[0]
    ) * (1.0 / s)


@functools.lru_cache(maxsize=None)
def _make_table_kernel(v_dim, d_dim, s_len):
    return pl.pallas_call(
        functools.partial(_table_body, s_len),
        out_shape=jax.ShapeDtypeStruct((v_dim, 1), jnp.float32),
        in_specs=[
            pl.BlockSpec(memory_space=pltpu.VMEM),
            pl.BlockSpec(memory_space=pltpu.VMEM),
            pl.BlockSpec(memory_space=pltpu.SMEM),
        ],
        out_specs=pl.BlockSpec(memory_space=pltpu.VMEM),
    )


_UNROLL = 8


@functools.lru_cache(maxsize=None)
def _make_sc_kernel(b_rows, s_len, v_dim):
    info = plsc.get_sparse_core_info()
    nc, ns, lanes = info.num_cores, info.num_subcores, info.num_lanes
    nw = nc * ns
    rows_per_w = b_rows // nw
    group = lanes
    n_groups = rows_per_w // group
    u = _UNROLL
    n_steps, rem = divmod(s_len, u)

    mesh = plsc.VectorSubcoreMesh(core_axis_name="c", subcore_axis_name="s")

    @functools.partial(
        pl.kernel,
        mesh=mesh,
        out_type=jax.ShapeDtypeStruct((b_rows,), jnp.float32),
        scratch_types=[
            pltpu.VMEM((rows_per_w * s_len,), jnp.int32),
            pltpu.VMEM((v_dim,), jnp.float32),
            pltpu.VMEM((rows_per_w,), jnp.float32),
        ],
        compiler_params=pltpu.CompilerParams(needs_layout_passes=False),
    )
    def sc_kernel(x_hbm, t_hbm, out_hbm, x_v, t_v, o_v):
        wid = lax.axis_index("s") * nc + lax.axis_index("c")
        pltpu.sync_copy(t_hbm, t_v)
        pltpu.sync_copy(
            x_hbm.at[pl.ds(wid * (rows_per_w * s_len), rows_per_w * s_len)],
            x_v,
        )
        lane_base = lax.iota(jnp.int32, lanes) * s_len
        out_idx = lax.iota(jnp.int32, lanes)
        zero = jnp.zeros((lanes,), jnp.float32)

        def do_group(g, carry):
            idx0 = g * (group * s_len) + lane_base

            def step(k, accs):
                s0 = k * u
                out = []
                for j in range(u):
                    xi = plsc.load_gather(x_v, [idx0 + (s0 + j)])
                    out.append(accs[j] + plsc.load_gather(t_v, [xi]))
                return tuple(out)

            accs = lax.fori_loop(0, n_steps, step, (zero,) * u)
            acc = accs[0]
            for j in range(1, u):
                acc = acc + accs[j]
            for j in range(rem):
                xi = plsc.load_gather(x_v, [idx0 + (n_steps * u + j)])
                acc = acc + plsc.load_gather(t_v, [xi])
            res = 1.0 / (1.0 + jnp.exp(-acc))
            plsc.store_scatter(o_v, [g * group + out_idx], res)
            return carry

        lax.fori_loop(0, n_groups, do_group, 0)
        pltpu.sync_copy(o_v, out_hbm.at[pl.ds(wid * rows_per_w, rows_per_w)])

    return sc_kernel


def kernel(x, emb, W, b):
    b_rows, s_len = x.shape
    v_dim, d_dim = emb.shape
    t = _make_table_kernel(v_dim, d_dim, s_len)(emb, W, b)
    out = _make_sc_kernel(b_rows, s_len, v_dim)(x.reshape(-1), t.reshape(-1))
    return out.reshape(b_rows, 1)
